# baseline (speedup 1.0000x reference)
import functools

import jax
import jax.numpy as jnp
from jax import lax
from jax.experimental import pallas as pl
from jax.experimental.pallas import tpu as pltpu
from jax.experimental.pallas import tpu_sc as plsc

C = 1000
L = 16


@functools.lru_cache(maxsize=None)
def _build(B1: int, B2: int):
    info = plsc.get_sparse_core_info()
    NC, NS = info.num_cores, info.num_subcores
    NW = NC * NS
    assert B1 % NW == 0 and L <= B2 <= 2 * L
    BPW = B1 // NW
    NITER = BPW // 3
    LEFT = BPW - 3 * NITER
    assert LEFT <= 2

    mesh = plsc.VectorSubcoreMesh(core_axis_name="c", subcore_axis_name="s")

    @functools.partial(
        pl.kernel,
        mesh=mesh,
        out_type=jax.ShapeDtypeStruct((B1, B2, C), jnp.float32),
        compiler_params=pltpu.CompilerParams(needs_layout_passes=False),
        scratch_types=[
            pltpu.VMEM((BPW * B2 + L,), jnp.int32),
            pltpu.VMEM((2, B2, C), jnp.float32),
            pltpu.VMEM((1, B2, C), jnp.float32),
            pltpu.SemaphoreType.DMA,
            pltpu.SemaphoreType.DMA,
        ],
    )
    def onehot(idx_hbm, out_hbm, idx_v, buf_a, buf_b, sem_a, sem_b):
        wid = lax.axis_index("s") * NC + lax.axis_index("c")
        b0 = wid * BPW
        pltpu.sync_copy(idx_hbm.at[pl.ds(b0 * B2, BPW * B2)],
                        idx_v.at[pl.ds(0, BPW * B2)])

        zeros = jnp.zeros((L,), jnp.float32)
        ones = jnp.ones((L,), jnp.float32)
        lanes = lax.iota(jnp.int32, L)
        rows1 = jnp.minimum(lanes + L, B2 - 1)
        mask1 = lanes < (B2 - L)

        def zrow(s, carry):
            for buf, slots in ((buf_a, 2), (buf_b, 1)):
                for t in range(slots):
                    for u in range(C // L):
                        buf[t, s, pl.ds(u * L, L)] = zeros
                    buf[t, s, pl.ds(C - L, L)] = zeros
            return carry
        lax.fori_loop(0, B2, zrow, 0)

        def set_vals(buf, slot, batch, val_vec):
            base = batch * B2
            sl = jnp.full((L,), slot, jnp.int32)
            c0 = idx_v[pl.ds(base, L)]
            plsc.store_scatter(buf, [sl, lanes, c0], val_vec)
            c1 = idx_v[pl.ds(base + L, L)]
            plsc.store_scatter(buf, [sl, rows1, c1], val_vec, mask=mask1)

        def dma_a(j):
            return pltpu.async_copy(
                buf_a, out_hbm.at[pl.ds(b0 + 3 * j, 2)], sem_a)

        def dma_b(j):
            return pltpu.async_copy(
                buf_b, out_hbm.at[pl.ds(b0 + 3 * j + 2, 1)], sem_b)

        cp_a = cp_b = None
        for j in range(NITER):
            if j >= 1:
                cp_a.wait()
                set_vals(buf_a, 0, 3 * (j - 1), zeros)
                set_vals(buf_a, 1, 3 * (j - 1) + 1, zeros)
            set_vals(buf_a, 0, 3 * j, ones)
            set_vals(buf_a, 1, 3 * j + 1, ones)
            cp_a = dma_a(j)
            if j >= 1:
                cp_b.wait()
                set_vals(buf_b, 0, 3 * (j - 1) + 2, zeros)
            set_vals(buf_b, 0, 3 * j + 2, ones)
            cp_b = dma_b(j)

        if LEFT == 2:
            cp_a.wait()
            set_vals(buf_a, 0, 3 * (NITER - 1), zeros)
            set_vals(buf_a, 1, 3 * (NITER - 1) + 1, zeros)
            set_vals(buf_a, 0, 3 * NITER, ones)
            set_vals(buf_a, 1, 3 * NITER + 1, ones)
            cp_a = pltpu.async_copy(
                buf_a, out_hbm.at[pl.ds(b0 + 3 * NITER, 2)], sem_a)
        elif LEFT == 1:
            cp_a.wait()
            set_vals(buf_a, 0, 3 * (NITER - 1), zeros)
            set_vals(buf_a, 1, 3 * (NITER - 1) + 1, zeros)
            set_vals(buf_a, 0, 3 * NITER, ones)
            cp_a = pltpu.async_copy(
                buf_a.at[pl.ds(0, 1)], out_hbm.at[pl.ds(b0 + 3 * NITER, 1)],
                sem_a)
        cp_a.wait()
        cp_b.wait()

    return onehot


def kernel(inputs):
    B1, B2 = inputs.shape
    flat = inputs.reshape(B1 * B2).astype(jnp.int32)
    return _build(B1, B2)(flat)


# speedup vs baseline: 1.0059x; 1.0059x over previous
import functools

import jax
import jax.numpy as jnp
from jax import lax
from jax.experimental import pallas as pl
from jax.experimental.pallas import tpu as pltpu
from jax.experimental.pallas import tpu_sc as plsc

C = 1000
L = 16
NBUF = 3


@functools.lru_cache(maxsize=None)
def _build(B1: int, B2: int):
    info = plsc.get_sparse_core_info()
    NC, NS = info.num_cores, info.num_subcores
    NW = NC * NS
    assert B1 % NW == 0 and L <= B2 <= 2 * L
    BPW = B1 // NW

    mesh = plsc.VectorSubcoreMesh(core_axis_name="c", subcore_axis_name="s")

    @functools.partial(
        pl.kernel,
        mesh=mesh,
        out_type=jax.ShapeDtypeStruct((B1, B2, C), jnp.float32),
        compiler_params=pltpu.CompilerParams(needs_layout_passes=False),
        scratch_types=(
            [pltpu.VMEM((BPW * B2 + L,), jnp.int32)]
            + [pltpu.VMEM((1, B2, C), jnp.float32) for _ in range(NBUF)]
            + [pltpu.SemaphoreType.DMA for _ in range(NBUF)]
        ),
    )
    def onehot(idx_hbm, out_hbm, idx_v, *scratch):
        bufs = scratch[:NBUF]
        sems = scratch[NBUF:]
        wid = lax.axis_index("s") * NC + lax.axis_index("c")
        b0 = wid * BPW
        pltpu.sync_copy(idx_hbm.at[pl.ds(b0 * B2, BPW * B2)],
                        idx_v.at[pl.ds(0, BPW * B2)])

        zeros = jnp.zeros((L,), jnp.float32)
        ones = jnp.ones((L,), jnp.float32)
        lanes = lax.iota(jnp.int32, L)
        zeros_i = jnp.zeros((L,), jnp.int32)
        mask1 = lanes < (B2 - L)

        def zrow(s, carry):
            for b in range(NBUF):
                for u in range(C // L):
                    bufs[b][0, s, pl.ds(u * L, L)] = zeros
                bufs[b][0, s, pl.ds(C - L, L)] = zeros
            return carry
        lax.fori_loop(0, B2, zrow, 0)

        def set_vals(buf, batch, val_vec):
            base = batch * B2
            c0 = idx_v[pl.ds(base, L)]
            plsc.store_scatter(buf, [zeros_i, lanes, c0], val_vec)
            c1 = idx_v[pl.ds(base + L, L)]
            rows1 = jnp.minimum(lanes + L, B2 - 1)
            plsc.store_scatter(buf, [zeros_i, rows1, c1], val_vec,
                               mask=mask1)

        copies = [None] * BPW
        for bb in range(BPW):
            b = bb % NBUF
            if bb >= NBUF:
                copies[bb - NBUF].wait()
                set_vals(bufs[b], bb - NBUF, zeros)
            set_vals(bufs[b], bb, ones)
            copies[bb] = pltpu.async_copy(
                bufs[b], out_hbm.at[pl.ds(b0 + bb, 1)], sems[b])
        for bb in range(max(0, BPW - NBUF), BPW):
            copies[bb].wait()

    return onehot


def kernel(inputs):
    B1, B2 = inputs.shape
    flat = inputs.reshape(B1 * B2).astype(jnp.int32)
    return _build(B1, B2)(flat)
